# Initial kernel scaffold; baseline (speedup 1.0000x reference)
import jax
import jax.numpy as jnp
from jax.experimental import pallas as pl


def kernel(events):
    raise NotImplementedError("write your pallas kernel here")


# speedup vs baseline: 1.4967x; 1.4967x over previous
import functools

import jax
import jax.numpy as jnp
from jax import lax
from jax.experimental import pallas as pl
from jax.experimental.pallas import tpu as pltpu
from jax.experimental.pallas import tpu_sc as plsc

N = 8388608
SENSOR_X, SENSOR_Y = 640, 480
XBINS, YBINS = 32, 24
NBINS = 2 * YBINS * XBINS
FX = XBINS / SENSOR_X
FY = YBINS / SENSOR_Y

L = 16
NW = 32
EV_PER_W = N // NW
CHUNK = 4096
CH_F = CHUNK * 4
N_CHUNK = EV_PER_W // CHUNK
VEC_PER_CHUNK = CHUNK // L
BPW = NBINS // NW

_mesh = plsc.VectorSubcoreMesh(core_axis_name="c", subcore_axis_name="s")
_params = pltpu.CompilerParams(needs_layout_passes=False)


@functools.partial(
    pl.kernel,
    out_type=jax.ShapeDtypeStruct((NW, NBINS), jnp.float32),
    mesh=_mesh,
    scratch_types=[
        pltpu.VMEM((CH_F,), jnp.float32),
        pltpu.VMEM((NBINS * L,), jnp.float32),
        pltpu.VMEM((NBINS,), jnp.float32),
    ],
    compiler_params=_params,
)
def _hist_stage1(ev_hbm, out_hbm, buf, hist16, histr):
    wid = lax.axis_index("s") * 2 + lax.axis_index("c")
    iota = lax.iota(jnp.int32, L)
    iota4 = iota * 4
    zeros16 = jnp.zeros((L,), jnp.float32)
    ones16 = jnp.ones((L,), jnp.float32)

    def zero_body(i, carry):
        hist16[pl.ds(i * L, L)] = zeros16
        return carry

    lax.fori_loop(0, NBINS, zero_body, 0)

    base_w = wid * (EV_PER_W * 4)

    def chunk_body(c, carry):
        pltpu.sync_copy(ev_hbm.at[pl.ds(base_w + c * CH_F, CH_F)], buf)

        def vec_body(i, acc):
            off = i * (4 * L)
            vx = plsc.load_gather(buf, [iota4 + off])
            vy = plsc.load_gather(buf, [iota4 + (off + 1)])
            vp = plsc.load_gather(buf, [iota4 + (off + 3)])
            xb = jnp.clip((vx * FX).astype(jnp.int32), 0, XBINS - 1)
            yb = jnp.clip((vy * FY).astype(jnp.int32), 0, YBINS - 1)
            b = yb * XBINS + xb
            b = jnp.where(vp > 0.0, b, b + YBINS * XBINS)
            plsc.addupdate_scatter(hist16, [b * L + iota], ones16)
            return acc

        lax.fori_loop(0, VEC_PER_CHUNK, vec_body, 0, unroll=4)
        return carry

    lax.fori_loop(0, N_CHUNK, chunk_body, 0)

    def red_body(g, carry):
        base = g * (L * L)
        acc = zeros16
        for lane in range(L):
            acc = acc + plsc.load_gather(hist16, [base + iota * L + lane])
        histr[pl.ds(g * L, L)] = acc
        return carry

    lax.fori_loop(0, NBINS // L, red_body, 0)
    pltpu.sync_copy(histr, out_hbm.at[wid])


@functools.partial(
    pl.kernel,
    out_type=jax.ShapeDtypeStruct((NBINS,), jnp.float32),
    mesh=_mesh,
    scratch_types=[
        pltpu.VMEM((NW * NBINS,), jnp.float32),
        pltpu.VMEM((BPW,), jnp.float32),
    ],
    compiler_params=_params,
)
def _hist_stage2(parts_hbm, out_hbm, tbl, outv):
    wid = lax.axis_index("s") * 2 + lax.axis_index("c")
    pltpu.sync_copy(parts_hbm, tbl)

    def tot_body(i, acc):
        return acc + tbl[pl.ds(i * L, L)]

    acc = lax.fori_loop(0, NW * NBINS // L, tot_body,
                        jnp.zeros((L,), jnp.float32), unroll=4)
    total = jnp.sum(acc)
    bits = lax.bitcast_convert_type(total, jnp.int32)
    x0 = lax.bitcast_convert_type(jnp.int32(0x7EF311C3) - bits, jnp.float32)
    x1 = x0 * (2.0 - total * x0)
    x2 = x1 * (2.0 - total * x1)
    x3 = x2 * (2.0 - total * x2)
    inv = jnp.where(total > 0.0, x3, 1.0)

    for k in range(BPW // L):
        def w_body(w, a):
            return a + tbl[pl.ds(w * NBINS + wid * BPW + k * L, L)]

        a = lax.fori_loop(0, NW, w_body, jnp.zeros((L,), jnp.float32))
        outv[pl.ds(k * L, L)] = a * inv

    pltpu.sync_copy(outv, out_hbm.at[pl.ds(wid * BPW, BPW)])


def kernel(events):
    ev_flat = events.reshape(-1)
    parts = _hist_stage1(ev_flat)
    hist = _hist_stage2(parts.reshape(-1))
    return hist.reshape(2, YBINS, XBINS)
